# baseline (speedup 1.0000x reference)
import functools
import jax
import jax.numpy as jnp
from jax.experimental import pallas as pl
from jax.experimental.pallas import tpu as pltpu

NSAMPLE = 32
TK = 512
N = 4096
C = 64


def _leaky(x):
    return jnp.where(x >= 0, x, 0.1 * x)


def _fmap_body(x_ref, w_ref, b_ref, o_ref):
    o_ref[...] = jnp.dot(x_ref[...], w_ref[...].T,
                         preferred_element_type=jnp.float32) + b_ref[...]


def _fmap(xT, W, b, tn=512):
    B = xT.shape[0]
    return pl.pallas_call(
        _fmap_body,
        grid=(B, N // tn),
        in_specs=[
            pl.BlockSpec((1, tn, C), lambda bb, t: (bb, t, 0)),
            pl.BlockSpec((C, C), lambda bb, t: (0, 0)),
            pl.BlockSpec((1, C), lambda bb, t: (0, 0)),
        ],
        out_specs=pl.BlockSpec((1, tn, C), lambda bb, t: (bb, t, 0)),
        out_shape=jax.ShapeDtypeStruct((B, N, C), jnp.float32),
    )(xT, W, b[None, :])


def _fmap_body3(x_ref, w_ref, b_ref, o_ref):
    o_ref[...] = (jnp.dot(x_ref[0], w_ref[...].T,
                          preferred_element_type=jnp.float32) + b_ref[...])[None]


def _corr_body(a_ref, b_ref, o_ref):
    o_ref[...] = jnp.dot(a_ref[0], b_ref[0].T,
                         preferred_element_type=jnp.float32)[None] * 0.125


def _corr(f1T, f2T, tn=256):
    B = f1T.shape[0]
    return pl.pallas_call(
        _corr_body,
        grid=(B, N // tn),
        in_specs=[
            pl.BlockSpec((1, tn, C), lambda bb, t: (bb, t, 0)),
            pl.BlockSpec((1, N, C), lambda bb, t: (bb, 0, 0)),
        ],
        out_specs=pl.BlockSpec((1, tn, N), lambda bb, t: (bb, t, 0)),
        out_shape=jax.ShapeDtypeStruct((B, N, N), jnp.float32),
    )(f1T, f2T)


def _wn_body(dx_ref, dy_ref, dz_ref, cv_ref, w12_ref, b12_ref, w3_ref, b3_ref,
             w4_ref, b4_ref, h_ref, m_ref, s_ref, m_acc, s_acc):
    t = pl.program_id(1)
    nt = pl.num_programs(1)
    dx, dy, dz, cv = dx_ref[0], dy_ref[0], dz_ref[0], cv_ref[0]
    w12 = w12_ref[...]
    b12 = b12_ref[...]
    w3 = w3_ref[...]
    b3 = b3_ref[...]
    w4 = w4_ref[...]
    b4 = b4_ref[...]
    hid = []
    for i in range(16):
        v = dx * w12[i, 0] + dy * w12[i, 1] + dz * w12[i, 2] + cv * w12[i, 3] \
            + b12[0, i]
        hid.append(jnp.maximum(v, 0.0))
    out8 = []
    for j in range(8):
        v = b3[0, j]
        acc = hid[0] * w3[j, 0]
        for i in range(1, 16):
            acc = acc + hid[i] * w3[j, i]
        out8.append(jnp.maximum(acc + v, 0.0))
    h = out8[0] * w4[0, 0]
    for j in range(1, 8):
        h = h + out8[j] * w4[0, j]
    h = jnp.maximum(h + b4[0, 0], 0.0)
    h_ref[...] = h[None]

    tile_max = jnp.max(h, axis=0, keepdims=True)

    @pl.when(t == 0)
    def _init():
        m_acc[...] = jnp.full_like(m_acc, -jnp.inf)
        s_acc[...] = jnp.zeros_like(s_acc)

    m_old = m_acc[0:1]
    m_new = jnp.maximum(m_old, tile_max)
    s_tile = jnp.sum(jnp.exp(h - m_new), axis=0, keepdims=True)
    s_new = s_acc[0:1] * jnp.exp(m_old - m_new) + s_tile
    m_acc[0:1] = m_new
    s_acc[0:1] = s_new

    @pl.when(t == nt - 1)
    def _fin():
        m_ref[...] = m_acc[0:1][None]
        s_ref[...] = s_acc[0:1][None]


def _weightnet(dx, dy, dz, cv, params, tn=256):
    B = dx.shape[0]
    w1, b1 = params['wn_w1'], params['wn_b1']
    w2, b2 = params['wn_w2'], params['wn_b2']
    w12 = w2 @ w1
    b12 = w2 @ b1 + b2
    spec_t = pl.BlockSpec((1, tn, TK), lambda bb, t: (bb, t, 0))
    spec_c = lambda r, c: pl.BlockSpec((r, c), lambda bb, t: (0, 0))
    h, m, s = pl.pallas_call(
        _wn_body,
        grid=(B, N // tn),
        in_specs=[spec_t, spec_t, spec_t, spec_t,
                  spec_c(16, 4), spec_c(1, 16), spec_c(8, 16), spec_c(1, 8),
                  spec_c(1, 8), spec_c(1, 1)],
        out_specs=[spec_t,
                   pl.BlockSpec((1, 1, TK), lambda bb, t: (bb, 0, 0)),
                   pl.BlockSpec((1, 1, TK), lambda bb, t: (bb, 0, 0))],
        out_shape=[jax.ShapeDtypeStruct((B, N, TK), jnp.float32),
                   jax.ShapeDtypeStruct((B, 1, TK), jnp.float32),
                   jax.ShapeDtypeStruct((B, 1, TK), jnp.float32)],
        scratch_shapes=[pltpu.VMEM((8, TK), jnp.float32),
                        pltpu.VMEM((8, TK), jnp.float32)],
    )(dx, dy, dz, cv, w12, b12[None, :], params['wn_w3'],
      params['wn_b3'][None, :], params['wn_w4'], params['wn_b4'][None, :])
    return h, m, s


def _tail_body(h_ref, m_ref, s_ref, t2_ref, p1_ref, x1_ref, posw_ref,
               posb_ref, *rest):
    n_mlp = (len(rest) - 1) // 2
    o_ref = rest[-1]
    tn = h_ref.shape[1]
    h = h_ref[0]
    g = jnp.exp(h - m_ref[0]) / s_ref[0]
    iota = jax.lax.broadcasted_iota(jnp.int32, (tn, TK), 1)
    slots = []
    gw = g
    for _ in range(NSAMPLE):
        mx = jnp.max(gw, axis=1, keepdims=True)
        cand = gw == mx
        slot = jnp.min(jnp.where(cand, iota, N), axis=1, keepdims=True)
        slots.append(slot)
        gw = jnp.where(iota == slot, -jnp.inf, gw)
    knn = jnp.concatenate(slots, axis=1)
    oh3 = (jax.lax.broadcasted_iota(jnp.int32, (tn, NSAMPLE, TK), 2)
           == knn[:, :, None]).astype(jnp.float32)
    oh = oh3.reshape(tn * NSAMPLE, TK)
    G = jnp.dot(oh, t2_ref[0], preferred_element_type=jnp.float32,
                precision=jax.lax.Precision.HIGHEST)
    G3 = G.reshape(tn, NSAMPLE, 2 * C)
    gp2 = G3[:, :, :C]
    gxyz = G3[:, :, C:C + 4]
    dir3 = (gxyz - x1_ref[0][:, None, :]).reshape(tn * NSAMPLE, 4)
    dirc = (jnp.dot(dir3, posw_ref[...], preferred_element_type=jnp.float32)
            + posb_ref[...]).reshape(tn, NSAMPLE, C)
    x = _leaky((gp2 + p1_ref[0][:, None, :]) + dirc)
    for li in range(n_mlp):
        w = rest[2 * li][...]
        b = rest[2 * li + 1][...]
        x2 = jnp.dot(x.reshape(tn * NSAMPLE, C), w.T,
                     preferred_element_type=jnp.float32) + b
        x = _leaky(x2).reshape(tn, NSAMPLE, C)
    o_ref[...] = jnp.max(x, axis=1)[None]


def _tail(h, m, s, t2, p1T, x1pad, posW, posb, mlps, tn=128):
    B = h.shape[0]
    posWp = jnp.concatenate([posW.T, jnp.zeros((1, C), jnp.float32)], axis=0)
    spec_c = lambda r, c: pl.BlockSpec((r, c), lambda bb, t: (0, 0))
    ins = [h, m, s, t2, p1T, x1pad, posWp, posb[None, :]]
    in_specs = [
        pl.BlockSpec((1, tn, TK), lambda bb, t: (bb, t, 0)),
        pl.BlockSpec((1, 1, TK), lambda bb, t: (bb, 0, 0)),
        pl.BlockSpec((1, 1, TK), lambda bb, t: (bb, 0, 0)),
        pl.BlockSpec((1, TK, 2 * C), lambda bb, t: (bb, 0, 0)),
        pl.BlockSpec((1, tn, C), lambda bb, t: (bb, t, 0)),
        pl.BlockSpec((1, tn, 4), lambda bb, t: (bb, t, 0)),
        spec_c(4, C), spec_c(1, C),
    ]
    for (w, b) in mlps:
        ins += [w, b[None, :]]
        in_specs += [spec_c(C, C), spec_c(1, C)]
    return pl.pallas_call(
        _tail_body,
        grid=(B, N // tn),
        in_specs=in_specs,
        out_specs=pl.BlockSpec((1, tn, C), lambda bb, t: (bb, t, 0)),
        out_shape=jax.ShapeDtypeStruct((B, N, C), jnp.float32),
    )(*ins)


def _cross(x1T, x2T, f1T, f2T, posW, posb, mlps, params):
    corr = _corr(f1T, f2T)
    cv, ci = jax.lax.top_k(corr, TK)
    vxyz = jax.vmap(lambda p, i: p[i])(x2T, ci)
    dx = vxyz[..., 0] - x1T[:, :, None, 0]
    dy = vxyz[..., 1] - x1T[:, :, None, 1]
    dz = vxyz[..., 2] - x1T[:, :, None, 2]
    h, m, s = _weightnet(dx, dy, dz, cv, params)
    B = f2T.shape[0]
    tab = jnp.concatenate(
        [f2T[:, :TK, :], x2T[:, :TK, :],
         jnp.zeros((B, TK, C - 3), jnp.float32)], axis=-1)
    x1pad = jnp.concatenate(
        [x1T, jnp.zeros(x1T.shape[:2] + (1,), jnp.float32)], axis=-1)
    return _tail(h, m, s, tab, f1T, x1pad, posW, posb, mlps)


def kernel(pc1, pc2, feat1, feat2, params):
    x1T = jnp.transpose(pc1, (0, 2, 1))
    x2T = jnp.transpose(pc2, (0, 2, 1))
    f1T = jnp.transpose(feat1, (0, 2, 1))
    f2T = jnp.transpose(feat2, (0, 2, 1))
    mlp1 = [(params['mlp1_0_w'], params['mlp1_0_b']),
            (params['mlp1_1_w'], params['mlp1_1_b'])]
    mlp2 = [(params['mlp2_0_w'], params['mlp2_0_b'])]

    fm11 = _fmap(f1T, params['t11_w'], params['t11_b'])
    fm22 = _fmap(f2T, params['t22_w'], params['t22_b'])
    fm12 = _fmap(f2T, params['t11_w'], params['t11_b'])
    fm21 = _fmap(f1T, params['t22_w'], params['t22_b'])

    o1 = _cross(x1T, x2T, fm11, fm22, params['pos1_w'], params['pos1_b'],
                mlp1, params)
    f1nT = _fmap(o1, params['t1_w'], params['t1_b'])
    o2 = _cross(x2T, x1T, fm12, fm21, params['pos1_w'], params['pos1_b'],
                mlp1, params)
    f2nT = _fmap(o2, params['t2_w'], params['t2_b'])
    f1fT = _cross(x1T, x2T, f1nT, f2nT, params['pos2_w'], params['pos2_b'],
                  mlp2, params)
    f1n = jnp.transpose(f1nT, (0, 2, 1))
    f2n = jnp.transpose(f2nT, (0, 2, 1))
    f1f = jnp.transpose(f1fT, (0, 2, 1))
    return (f1n, f2n, f1f)


# speedup vs baseline: 1.0225x; 1.0225x over previous
import functools
import jax
import jax.numpy as jnp
from jax.experimental import pallas as pl
from jax.experimental.pallas import tpu as pltpu

NSAMPLE = 32
TK = 512
N = 4096
C = 64


def _leaky(x):
    return jnp.where(x >= 0, x, 0.1 * x)


def _fmap_body(x_ref, w_ref, b_ref, o_ref):
    o_ref[...] = jnp.dot(x_ref[...], w_ref[...].T,
                         preferred_element_type=jnp.float32) + b_ref[...]


def _fmap(xT, W, b, tn=512):
    B = xT.shape[0]
    return pl.pallas_call(
        _fmap_body,
        grid=(B, N // tn),
        in_specs=[
            pl.BlockSpec((1, tn, C), lambda bb, t: (bb, t, 0)),
            pl.BlockSpec((C, C), lambda bb, t: (0, 0)),
            pl.BlockSpec((1, C), lambda bb, t: (0, 0)),
        ],
        out_specs=pl.BlockSpec((1, tn, C), lambda bb, t: (bb, t, 0)),
        out_shape=jax.ShapeDtypeStruct((B, N, C), jnp.float32),
    )(xT, W, b[None, :])


def _fmap_body3(x_ref, w_ref, b_ref, o_ref):
    o_ref[...] = (jnp.dot(x_ref[0], w_ref[...].T,
                          preferred_element_type=jnp.float32) + b_ref[...])[None]


def _corr_body(a_ref, b_ref, o_ref):
    o_ref[...] = jnp.dot(a_ref[0], b_ref[0].T,
                         preferred_element_type=jnp.float32)[None] * 0.125


def _corr(f1T, f2T, tn=256):
    B = f1T.shape[0]
    return pl.pallas_call(
        _corr_body,
        grid=(B, N // tn),
        in_specs=[
            pl.BlockSpec((1, tn, C), lambda bb, t: (bb, t, 0)),
            pl.BlockSpec((1, N, C), lambda bb, t: (bb, 0, 0)),
        ],
        out_specs=pl.BlockSpec((1, tn, N), lambda bb, t: (bb, t, 0)),
        out_shape=jax.ShapeDtypeStruct((B, N, N), jnp.float32),
    )(f1T, f2T)


def _bitonic_topk(v, idx, width, k_out):
    rr = v.shape[0]
    iota = jax.lax.broadcasted_iota(jnp.int32, (rr, width), 1)
    nlev = width.bit_length() - 1
    for k_exp in range(1, nlev + 1):
        k = 1 << k_exp
        for j_exp in range(k_exp - 1, -1, -1):
            d = 1 << j_exp
            bit = (iota & d) != 0
            pv = jnp.where(bit, jnp.roll(v, d, 1), jnp.roll(v, -d, 1))
            pi = jnp.where(bit, jnp.roll(idx, d, 1), jnp.roll(idx, -d, 1))
            desc_blk = (iota & k) == 0
            keep_max = desc_blk == (~bit)
            gt = (v > pv) | ((v == pv) & (idx < pi))
            sel = gt == keep_max
            v = jnp.where(sel, v, pv)
            idx = jnp.where(sel, idx, pi)
    return v[:, :k_out], idx[:, :k_out]


def _corr_topk_body(a_ref, b_ref, cv_ref, ci_ref):
    v = jnp.dot(a_ref[0], b_ref[0].T,
                preferred_element_type=jnp.float32) * 0.125
    idx = jax.lax.broadcasted_iota(jnp.int32, v.shape, 1)
    cv, ci = _bitonic_topk(v, idx, N, TK)
    cv_ref[...] = cv[None]
    ci_ref[...] = ci[None]


def _corr_topk(f1T, f2T, tn=256):
    B = f1T.shape[0]
    return pl.pallas_call(
        _corr_topk_body,
        grid=(B, N // tn),
        in_specs=[
            pl.BlockSpec((1, tn, C), lambda bb, t: (bb, t, 0)),
            pl.BlockSpec((1, N, C), lambda bb, t: (bb, 0, 0)),
        ],
        out_specs=[pl.BlockSpec((1, tn, TK), lambda bb, t: (bb, t, 0)),
                   pl.BlockSpec((1, tn, TK), lambda bb, t: (bb, t, 0))],
        out_shape=[jax.ShapeDtypeStruct((B, N, TK), jnp.float32),
                   jax.ShapeDtypeStruct((B, N, TK), jnp.int32)],
    )(f1T, f2T)


def _wn_body(dx_ref, dy_ref, dz_ref, cv_ref, w12_ref, b12_ref, w3_ref, b3_ref,
             w4_ref, b4_ref, h_ref, m_ref, s_ref, m_acc, s_acc):
    t = pl.program_id(1)
    nt = pl.num_programs(1)
    dx, dy, dz, cv = dx_ref[0], dy_ref[0], dz_ref[0], cv_ref[0]
    w12 = w12_ref[...]
    b12 = b12_ref[...]
    w3 = w3_ref[...]
    b3 = b3_ref[...]
    w4 = w4_ref[...]
    b4 = b4_ref[...]
    hid = []
    for i in range(16):
        v = dx * w12[i, 0] + dy * w12[i, 1] + dz * w12[i, 2] + cv * w12[i, 3] \
            + b12[0, i]
        hid.append(jnp.maximum(v, 0.0))
    out8 = []
    for j in range(8):
        v = b3[0, j]
        acc = hid[0] * w3[j, 0]
        for i in range(1, 16):
            acc = acc + hid[i] * w3[j, i]
        out8.append(jnp.maximum(acc + v, 0.0))
    h = out8[0] * w4[0, 0]
    for j in range(1, 8):
        h = h + out8[j] * w4[0, j]
    h = jnp.maximum(h + b4[0, 0], 0.0)
    h_ref[...] = h[None]

    tile_max = jnp.max(h, axis=0, keepdims=True)

    @pl.when(t == 0)
    def _init():
        m_acc[...] = jnp.full_like(m_acc, -jnp.inf)
        s_acc[...] = jnp.zeros_like(s_acc)

    m_old = m_acc[0:1]
    m_new = jnp.maximum(m_old, tile_max)
    s_tile = jnp.sum(jnp.exp(h - m_new), axis=0, keepdims=True)
    s_new = s_acc[0:1] * jnp.exp(m_old - m_new) + s_tile
    m_acc[0:1] = m_new
    s_acc[0:1] = s_new

    @pl.when(t == nt - 1)
    def _fin():
        m_ref[...] = m_acc[0:1][None]
        s_ref[...] = s_acc[0:1][None]


def _weightnet(dx, dy, dz, cv, params, tn=256):
    B = dx.shape[0]
    w1, b1 = params['wn_w1'], params['wn_b1']
    w2, b2 = params['wn_w2'], params['wn_b2']
    w12 = w2 @ w1
    b12 = w2 @ b1 + b2
    spec_t = pl.BlockSpec((1, tn, TK), lambda bb, t: (bb, t, 0))
    spec_c = lambda r, c: pl.BlockSpec((r, c), lambda bb, t: (0, 0))
    h, m, s = pl.pallas_call(
        _wn_body,
        grid=(B, N // tn),
        in_specs=[spec_t, spec_t, spec_t, spec_t,
                  spec_c(16, 4), spec_c(1, 16), spec_c(8, 16), spec_c(1, 8),
                  spec_c(1, 8), spec_c(1, 1)],
        out_specs=[spec_t,
                   pl.BlockSpec((1, 1, TK), lambda bb, t: (bb, 0, 0)),
                   pl.BlockSpec((1, 1, TK), lambda bb, t: (bb, 0, 0))],
        out_shape=[jax.ShapeDtypeStruct((B, N, TK), jnp.float32),
                   jax.ShapeDtypeStruct((B, 1, TK), jnp.float32),
                   jax.ShapeDtypeStruct((B, 1, TK), jnp.float32)],
        scratch_shapes=[pltpu.VMEM((8, TK), jnp.float32),
                        pltpu.VMEM((8, TK), jnp.float32)],
    )(dx, dy, dz, cv, w12, b12[None, :], params['wn_w3'],
      params['wn_b3'][None, :], params['wn_w4'], params['wn_b4'][None, :])
    return h, m, s


def _tail_body(knn_ref, t2_ref, p1_ref, x1_ref, posw_ref,
               posb_ref, *rest):
    n_mlp = (len(rest) - 1) // 2
    o_ref = rest[-1]
    tn = knn_ref.shape[1]
    knn = knn_ref[0]
    oh3 = (jax.lax.broadcasted_iota(jnp.int32, (tn, NSAMPLE, TK), 2)
           == knn[:, :, None]).astype(jnp.float32)
    oh = oh3.reshape(tn * NSAMPLE, TK)
    G = jnp.dot(oh, t2_ref[0], preferred_element_type=jnp.float32,
                precision=jax.lax.Precision.HIGHEST)
    G3 = G.reshape(tn, NSAMPLE, 2 * C)
    gp2 = G3[:, :, :C]
    gxyz = G3[:, :, C:C + 4]
    dir3 = (gxyz - x1_ref[0][:, None, :]).reshape(tn * NSAMPLE, 4)
    dirc = (jnp.dot(dir3, posw_ref[...], preferred_element_type=jnp.float32)
            + posb_ref[...]).reshape(tn, NSAMPLE, C)
    x = _leaky((gp2 + p1_ref[0][:, None, :]) + dirc)
    for li in range(n_mlp):
        w = rest[2 * li][...]
        b = rest[2 * li + 1][...]
        x2 = jnp.dot(x.reshape(tn * NSAMPLE, C), w.T,
                     preferred_element_type=jnp.float32) + b
        x = _leaky(x2).reshape(tn, NSAMPLE, C)
    o_ref[...] = jnp.max(x, axis=1)[None]


def _tail(knn, t2, p1T, x1pad, posW, posb, mlps, tn=128):
    B = knn.shape[0]
    posWp = jnp.concatenate([posW.T, jnp.zeros((1, C), jnp.float32)], axis=0)
    spec_c = lambda r, c: pl.BlockSpec((r, c), lambda bb, t: (0, 0))
    ins = [knn, t2, p1T, x1pad, posWp, posb[None, :]]
    in_specs = [
        pl.BlockSpec((1, tn, NSAMPLE), lambda bb, t: (bb, t, 0)),
        pl.BlockSpec((1, TK, 2 * C), lambda bb, t: (bb, 0, 0)),
        pl.BlockSpec((1, tn, C), lambda bb, t: (bb, t, 0)),
        pl.BlockSpec((1, tn, 4), lambda bb, t: (bb, t, 0)),
        spec_c(4, C), spec_c(1, C),
    ]
    for (w, b) in mlps:
        ins += [w, b[None, :]]
        in_specs += [spec_c(C, C), spec_c(1, C)]
    return pl.pallas_call(
        _tail_body,
        grid=(B, N // tn),
        in_specs=in_specs,
        out_specs=pl.BlockSpec((1, tn, C), lambda bb, t: (bb, t, 0)),
        out_shape=jax.ShapeDtypeStruct((B, N, C), jnp.float32),
    )(*ins)


def _cross(x1T, x2T, f1T, f2T, posW, posb, mlps, params):
    B = f2T.shape[0]
    cv, ci = _corr_topk(f1T, f2T)
    valid_xyz = jax.vmap(lambda p, i: p[i])(x2T, ci) - x1T[:, :, None, :]
    inp = jnp.concatenate([valid_xyz, cv.reshape(B, N, TK, 1)], axis=-1)
    hh = inp @ params['wn_w1'].T + params['wn_b1']
    hh = hh @ params['wn_w2'].T + params['wn_b2']
    hh = jax.nn.relu(hh)
    hh = hh @ params['wn_w3'].T + params['wn_b3']
    hh = jax.nn.relu(hh)
    hh = hh @ params['wn_w4'].T + params['wn_b4']
    hh = jax.nn.relu(hh)
    hh = hh.reshape(B, N, TK)
    hh = jax.nn.softmax(hh, axis=1)
    _, knn = jax.lax.top_k(hh, NSAMPLE)
    tab = jnp.concatenate(
        [f2T[:, :TK, :], x2T[:, :TK, :],
         jnp.zeros((B, TK, C - 3), jnp.float32)], axis=-1)
    x1pad = jnp.concatenate(
        [x1T, jnp.zeros(x1T.shape[:2] + (1,), jnp.float32)], axis=-1)
    return _tail(knn, tab, f1T, x1pad, posW, posb, mlps)


def kernel(pc1, pc2, feat1, feat2, params):
    x1T = jnp.transpose(pc1, (0, 2, 1))
    x2T = jnp.transpose(pc2, (0, 2, 1))
    f1T = jnp.transpose(feat1, (0, 2, 1))
    f2T = jnp.transpose(feat2, (0, 2, 1))
    mlp1 = [(params['mlp1_0_w'], params['mlp1_0_b']),
            (params['mlp1_1_w'], params['mlp1_1_b'])]
    mlp2 = [(params['mlp2_0_w'], params['mlp2_0_b'])]

    fm11 = _fmap(f1T, params['t11_w'], params['t11_b'])
    fm22 = _fmap(f2T, params['t22_w'], params['t22_b'])
    fm12 = _fmap(f2T, params['t11_w'], params['t11_b'])
    fm21 = _fmap(f1T, params['t22_w'], params['t22_b'])

    o1 = _cross(x1T, x2T, fm11, fm22, params['pos1_w'], params['pos1_b'],
                mlp1, params)
    f1nT = _fmap(o1, params['t1_w'], params['t1_b'])
    o2 = _cross(x2T, x1T, fm12, fm21, params['pos1_w'], params['pos1_b'],
                mlp1, params)
    f2nT = _fmap(o2, params['t2_w'], params['t2_b'])
    f1fT = _cross(x1T, x2T, f1nT, f2nT, params['pos2_w'], params['pos2_b'],
                  mlp2, params)
    f1n = jnp.transpose(f1nT, (0, 2, 1))
    f2n = jnp.transpose(f2nT, (0, 2, 1))
    f1f = jnp.transpose(f1fT, (0, 2, 1))
    return (f1n, f2n, f1f)
